# baseline (speedup 1.0000x reference)
import jax
import jax.numpy as jnp
from jax import lax
from jax.experimental import pallas as pl
from jax.experimental.pallas import tpu as pltpu
from jax.experimental.pallas import tpu_sc as plsc

_BLK = 1024
_BT = 256
_NC = 2
_NS = 16
_NW = _NC * _NS


def kernel(position_idx, table):
    batch, seq = position_idx.shape
    n = batch * seq
    dim = table.shape[1]
    idx = position_idx.reshape(1, n)

    per_w = n // _NW
    nblk = per_w // _BLK

    mesh = plsc.VectorSubcoreMesh(core_axis_name="core",
                                  subcore_axis_name="subcore")

    @jax.jit
    def run(table_arr, idx_arr):
        @pl.kernel(out_type=jax.ShapeDtypeStruct((n, dim),
                                                 table_arr.dtype),
                   mesh=mesh,
                   scratch_types=[
                       pltpu.VMEM((2, 1, _BLK), jnp.int32),
                       pltpu.VMEM((2, _BLK, dim), jnp.float32),
                       pltpu.SemaphoreType.DMA,
                       pltpu.SemaphoreType.DMA,
                       pltpu.SemaphoreType.DMA,
                   ],
                   compiler_params=pltpu.CompilerParams(
                       use_tc_tiling_on_sc=False))
        def gather_kernel(table_hbm, idx_hbm, out_hbm, idx_v, out_v,
                          sem_g, sem_o0, sem_o1):
            wid = lax.axis_index("subcore") * _NC + lax.axis_index("core")
            base = wid * per_w
            sems = (sem_o0, sem_o1)

            @pl.loop(0, nblk, step=2)
            def _(i):
                for r in range(2):
                    b = i + r
                    off = base + b * _BLK

                    @pl.when(b >= 2)
                    def _():
                        pltpu.make_async_copy(
                            out_v.at[r],
                            out_hbm.at[pl.ds(off - 2 * _BLK, _BLK)],
                            sems[r],
                        ).wait()

                    pltpu.sync_copy(idx_hbm.at[0, pl.ds(off, _BLK)],
                                    idx_v.at[r, 0])

                    pltpu.async_copy(
                        table_hbm.at[idx_v.at[r, 0]],
                        out_v.at[r],
                        sem_g,
                    ).wait()

                    pltpu.async_copy(out_v.at[r],
                                     out_hbm.at[pl.ds(off, _BLK)],
                                     sems[r])

            for r in range(2):
                last_off = base + (nblk - 2 + r) * _BLK
                pltpu.make_async_copy(
                    out_v.at[r],
                    out_hbm.at[pl.ds(last_off, _BLK)],
                    sems[r],
                ).wait()

        return gather_kernel(table_arr, idx_arr)

    flat = run(table, idx)
    pack = 128 // dim
    njt = seq * dim // 128
    g = flat.reshape(n // pack, 128)

    def _transpose_body(g_ref, o_ref):
        x3 = g_ref[...].reshape(_BT, njt, 128)
        for j in range(njt):
            o_ref[j] = x3[:, j, :].T

    out3 = pl.pallas_call(
        _transpose_body,
        grid=(batch // _BT,),
        in_specs=[pl.BlockSpec((_BT * njt, 128), lambda i: (i, 0))],
        out_specs=pl.BlockSpec((njt, 128, _BT), lambda i: (0, 0, i)),
        out_shape=jax.ShapeDtypeStruct((njt, 128, batch), jnp.float32),
        compiler_params=pltpu.CompilerParams(
            dimension_semantics=("parallel",)),
    )(g)

    return out3.reshape(seq, dim, batch).transpose(2, 0, 1)


# speedup vs baseline: 1.0145x; 1.0145x over previous
import jax
import jax.numpy as jnp
from jax import lax
from jax.experimental import pallas as pl
from jax.experimental.pallas import tpu as pltpu
from jax.experimental.pallas import tpu_sc as plsc

_BLK = 1024
_BT = 512
_NC = 2
_NS = 16
_NW = _NC * _NS


def kernel(position_idx, table):
    batch, seq = position_idx.shape
    n = batch * seq
    dim = table.shape[1]
    idx = position_idx.reshape(1, n)

    per_w = n // _NW
    nblk = per_w // _BLK

    mesh = plsc.VectorSubcoreMesh(core_axis_name="core",
                                  subcore_axis_name="subcore")

    @jax.jit
    def run(table_arr, idx_arr):
        @pl.kernel(out_type=jax.ShapeDtypeStruct((n, dim),
                                                 table_arr.dtype),
                   mesh=mesh,
                   scratch_types=[
                       pltpu.VMEM((2, 1, _BLK), jnp.int32),
                       pltpu.VMEM((2, _BLK, dim), jnp.float32),
                       pltpu.SemaphoreType.DMA,
                       pltpu.SemaphoreType.DMA,
                       pltpu.SemaphoreType.DMA,
                   ],
                   compiler_params=pltpu.CompilerParams(
                       use_tc_tiling_on_sc=False))
        def gather_kernel(table_hbm, idx_hbm, out_hbm, idx_v, out_v,
                          sem_g, sem_o0, sem_o1):
            wid = lax.axis_index("subcore") * _NC + lax.axis_index("core")
            base = wid * per_w
            sems = (sem_o0, sem_o1)

            @pl.loop(0, nblk, step=2)
            def _(i):
                for r in range(2):
                    b = i + r
                    off = base + b * _BLK

                    @pl.when(b >= 2)
                    def _():
                        pltpu.make_async_copy(
                            out_v.at[r],
                            out_hbm.at[pl.ds(off - 2 * _BLK, _BLK)],
                            sems[r],
                        ).wait()

                    pltpu.sync_copy(idx_hbm.at[0, pl.ds(off, _BLK)],
                                    idx_v.at[r, 0])

                    pltpu.async_copy(
                        table_hbm.at[idx_v.at[r, 0]],
                        out_v.at[r],
                        sem_g,
                    ).wait()

                    pltpu.async_copy(out_v.at[r],
                                     out_hbm.at[pl.ds(off, _BLK)],
                                     sems[r])

            for r in range(2):
                last_off = base + (nblk - 2 + r) * _BLK
                pltpu.make_async_copy(
                    out_v.at[r],
                    out_hbm.at[pl.ds(last_off, _BLK)],
                    sems[r],
                ).wait()

        return gather_kernel(table_arr, idx_arr)

    flat = run(table, idx)
    pack = 128 // dim
    njt = seq * dim // 128
    g = flat.reshape(n // pack, 128)

    def _transpose_body(g_ref, o_ref):
        x3 = g_ref[...].reshape(_BT, njt, 128)
        for j in range(njt):
            o_ref[j] = x3[:, j, :].T

    out3 = pl.pallas_call(
        _transpose_body,
        grid=(batch // _BT,),
        in_specs=[pl.BlockSpec((_BT * njt, 128), lambda i: (i, 0))],
        out_specs=pl.BlockSpec((njt, 128, _BT), lambda i: (0, 0, i)),
        out_shape=jax.ShapeDtypeStruct((njt, 128, batch), jnp.float32),
        compiler_params=pltpu.CompilerParams(
            dimension_semantics=("parallel",)),
    )(g)

    return out3.reshape(seq, dim, batch).transpose(2, 0, 1)


# speedup vs baseline: 1.0830x; 1.0676x over previous
import jax
import jax.numpy as jnp
from jax import lax
from jax.experimental import pallas as pl
from jax.experimental.pallas import tpu as pltpu
from jax.experimental.pallas import tpu_sc as plsc

_BROWS = 16
_BT = 512
_NC = 2
_NS = 16
_NW = _NC * _NS
_K = 2


def kernel(position_idx, table):
    batch, seq = position_idx.shape
    dim = table.shape[1]
    chunks = [(0, 96), (96, 96), (192, 8)]
    per_b = batch // _NW
    nblk = per_b // _BROWS

    mesh = plsc.VectorSubcoreMesh(core_axis_name="core",
                                  subcore_axis_name="subcore")

    def gather_chunk(table_arr, idx_arr, s0, ns):
        nk = batch * ns
        @pl.kernel(out_type=jax.ShapeDtypeStruct((nk, dim),
                                                 table_arr.dtype),
                   mesh=mesh,
                   scratch_types=[
                       pltpu.VMEM((2, _BROWS, ns), jnp.int32),
                       pltpu.VMEM((2, _BROWS * ns, dim), jnp.float32),
                       pltpu.SemaphoreType.DMA,
                       pltpu.SemaphoreType.DMA,
                       pltpu.SemaphoreType.DMA,
                   ],
                   compiler_params=pltpu.CompilerParams(
                       use_tc_tiling_on_sc=False))
        def gather_kernel(table_hbm, idx_hbm, out_hbm, idx_v, out_v,
                          sem_g, sem_o0, sem_o1):
            wid = lax.axis_index("subcore") * _NC + lax.axis_index("core")
            b_base = wid * per_b
            sems = (sem_o0, sem_o1)
            blk_n = _BROWS * ns

            @pl.loop(0, nblk, step=2)
            def _(i):
                for r in range(2):
                    blk = i + r
                    b0 = b_base + blk * _BROWS
                    off = b0 * ns

                    @pl.when(blk >= 2)
                    def _():
                        pltpu.make_async_copy(
                            out_v.at[r],
                            out_hbm.at[pl.ds(off - 2 * blk_n, blk_n)],
                            sems[r],
                        ).wait()

                    pltpu.sync_copy(
                        idx_hbm.at[pl.ds(b0, _BROWS), pl.ds(s0, ns)],
                        idx_v.at[r])

                    copies = [
                        pltpu.async_copy(
                            table_hbm.at[idx_v.at[r, row]],
                            out_v.at[r, pl.ds(row * ns, ns)],
                            sem_g,
                        )
                        for row in range(_BROWS)
                    ]
                    for c in copies:
                        c.wait()

                    pltpu.async_copy(out_v.at[r],
                                     out_hbm.at[pl.ds(off, blk_n)],
                                     sems[r])

            for r in range(2):
                last = (b_base + (nblk - 2 + r) * _BROWS) * ns
                pltpu.make_async_copy(
                    out_v.at[r],
                    out_hbm.at[pl.ds(last, blk_n)],
                    sems[r],
                ).wait()

        return gather_kernel(table_arr, idx_arr)

    def make_transpose_body(njt):
        def transpose_body(g_ref, o_ref):
            x3 = g_ref[...].reshape(_BT, njt, 128)
            for j in range(njt):
                o_ref[j] = x3[:, j, :].T
        return transpose_body

    def make_transpose_body_alias(njt):
        body = make_transpose_body(njt)
        def transpose_body_alias(g_ref, buf_ref, o_ref):
            del buf_ref
            body(g_ref, o_ref)
        return transpose_body_alias

    njt_total = seq * dim // 128
    out_shape = jax.ShapeDtypeStruct((njt_total, 128, batch), jnp.float32)

    @jax.jit
    def run(table_arr, idx_arr):
        buf = None
        for s0, ns in chunks:
            njt = ns * dim // 128
            j0 = s0 * dim // 128
            flat = gather_chunk(table_arr, idx_arr, s0, ns)
            g = flat.reshape(batch * ns * dim // 128, 128)
            if buf is None:
                buf = pl.pallas_call(
                    make_transpose_body(njt),
                    grid=(batch // _BT,),
                    in_specs=[pl.BlockSpec((_BT * njt, 128),
                                           lambda i: (i, 0))],
                    out_specs=pl.BlockSpec(
                        (njt, 128, _BT),
                        lambda i, j0=j0, njt=njt: (j0 // njt, 0, i)),
                    out_shape=out_shape,
                    compiler_params=pltpu.CompilerParams(
                        dimension_semantics=("parallel",)),
                )(g)
            else:
                buf = pl.pallas_call(
                    make_transpose_body_alias(njt),
                    grid=(batch // _BT,),
                    in_specs=[
                        pl.BlockSpec((_BT * njt, 128), lambda i: (i, 0)),
                        pl.BlockSpec(memory_space=pl.ANY),
                    ],
                    out_specs=pl.BlockSpec(
                        (njt, 128, _BT),
                        lambda i, j0=j0, njt=njt: (j0 // njt, 0, i)),
                    out_shape=out_shape,
                    input_output_aliases={1: 0},
                    compiler_params=pltpu.CompilerParams(
                        dimension_semantics=("parallel",)),
                )(g, buf)
        return buf

    out3 = run(table, position_idx)
    return out3.reshape(seq, dim, batch).transpose(2, 0, 1)
